# baseline (speedup 1.0000x reference)
import functools

import jax
import jax.numpy as jnp
from jax import lax
from jax.experimental import pallas as pl
from jax.experimental.pallas import tpu as pltpu
from jax.experimental.pallas import tpu_sc as plsc

_NC = 2
_NS = 16
_NW = _NC * _NS
_GROUP = 128
_DC = 32


def _proj_body(feat_ref, wn_ref, ws_ref, b_ref, hext_ref, self_ref):
    f = feat_ref[...]
    dn = (((1,), (1,)), ((), ()))
    h = lax.dot_general(f, wn_ref[...], dn, preferred_element_type=jnp.float32)
    d = h.shape[1]
    hext_ref[:, :d] = h.astype(jnp.bfloat16)
    hext_ref[:, d:] = jnp.ones((f.shape[0], _DC), jnp.bfloat16)
    s = lax.dot_general(f, ws_ref[...], dn, preferred_element_type=jnp.float32)
    self_ref[...] = s + b_ref[...]


def _combine_body(n, d, parts_ref, self_ref, out_ref):
    p0 = parts_ref[0]
    summed = p0[:n, :d].astype(jnp.float32)
    cnt = p0[:n, d:d + 1].astype(jnp.float32)
    out_ref[...] = self_ref[...] + summed / jnp.maximum(cnt, 1.0)


def _sc_agg_body(n_acc, de, g0, g1, hext_hbm, edges_hbm,
                 out_hbm, ebuf, rows, zbuf, acc, ei0, ei1, ei2, ei3, ro0, ro1):
    cid = lax.axis_index("c")
    sid = lax.axis_index("s")
    rows_per_tile = n_acc // _NS
    row0 = sid * rows_per_tile
    ei = (ei0, ei1, ei2, ei3)
    ro = (ro0, ro1)
    gstart = sid * g0
    groups = g0

    for q in range(4):
        pltpu.async_copy(edges_hbm.at[gstart + q], ebuf.at[q], ei[q])

    zrows = zbuf.shape[0]

    @pl.loop(0, zrows)
    def _(i):
        @pl.loop(0, de, step=32)
        def _(j):
            zbuf[i, pl.ds(j, 32)] = jnp.zeros((32,), jnp.bfloat16)

    @pl.loop(0, rows_per_tile, step=zrows)
    def _(r):
        pltpu.sync_copy(zbuf, acc.at[pl.ds(row0 + r, zrows)])

    plsc.subcore_barrier()

    for p in range(2):
        pltpu.make_async_copy(edges_hbm.at[gstart + p], ebuf.at[p],
                              ei[p]).wait()
        pltpu.async_copy(hext_hbm.at[ebuf.at[p, 0]], rows.at[p], ro[p])

    @pl.loop(0, groups, step=4)
    def _(g):
        for k in range(4):
            gi = g + k
            p, q = k % 2, k
            pltpu.make_async_copy(hext_hbm.at[ebuf.at[q, 0]], rows.at[p],
                                  ro[p]).wait()
            pltpu.sync_copy(rows.at[p], acc.at[ebuf.at[q, 1]], add=True)

            @pl.when(gi + 2 < groups)
            def _():
                q2 = (q + 2) % 4
                pltpu.make_async_copy(edges_hbm.at[gstart + gi + 2],
                                      ebuf.at[q2], ei[q2]).wait()
                pltpu.async_copy(hext_hbm.at[ebuf.at[q2, 0]], rows.at[p],
                                 ro[p])

            @pl.when(gi + 4 < groups)
            def _():
                pltpu.async_copy(edges_hbm.at[gstart + gi + 4],
                                 ebuf.at[q], ei[q])

    plsc.subcore_barrier()
    pltpu.sync_copy(acc.at[pl.ds(row0, rows_per_tile)],
                    out_hbm.at[cid, pl.ds(row0, rows_per_tile)])


def kernel(feat, edge_index, W_neigh, W_self, b_self):
    n, d_in = feat.shape
    d_out = W_neigh.shape[0]
    e = edge_index.shape[1]
    de = d_out + _DC

    g0 = -(-e // (_NS * _GROUP))
    g0 = -(-g0 // 4) * 4
    g1 = 0
    total_groups = _NS * g0
    e_pad = total_groups * _GROUP
    n_acc = -(-(n + 1) // (_NS * 8)) * (_NS * 8)

    src = edge_index[0]
    dst = edge_index[1]
    pad = e_pad - e
    if pad:
        src = jnp.concatenate([src, jnp.zeros((pad,), jnp.int32)])
        dst = jnp.concatenate([dst, jnp.full((pad,), n, jnp.int32)])
    edges = jnp.stack([src.reshape(total_groups, _GROUP),
                       dst.reshape(total_groups, _GROUP)], axis=1)

    h_ext, rst_self = pl.pallas_call(
        _proj_body,
        out_shape=[
            jax.ShapeDtypeStruct((n, de), jnp.bfloat16),
            jax.ShapeDtypeStruct((n, d_out), jnp.float32),
        ],
    )(feat, W_neigh, W_self, b_self.reshape(1, d_out))

    sc_agg = pl.kernel(
        functools.partial(_sc_agg_body, n_acc, de, g0, g1),
        out_type=jax.ShapeDtypeStruct((1, n_acc, de), jnp.bfloat16),
        mesh=plsc.VectorSubcoreMesh(core_axis_name="c", subcore_axis_name="s",
                                    num_cores=1),
        compiler_params=pltpu.CompilerParams(use_tc_tiling_on_sc=False),
        scratch_types=[
            pltpu.VMEM((4, 2, _GROUP), jnp.int32),
            pltpu.VMEM((2, _GROUP, de), jnp.bfloat16),
            pltpu.VMEM((n_acc // _NS // 4, de), jnp.bfloat16),
            pltpu.VMEM_SHARED((n_acc, de), jnp.bfloat16),
        ] + [pltpu.SemaphoreType.DMA] * 6,
    )
    partials = sc_agg(h_ext, edges)

    rst = pl.pallas_call(
        functools.partial(_combine_body, n, d_out),
        out_shape=jax.ShapeDtypeStruct((n, d_out), jnp.float32),
    )(partials, rst_self)
    return rst


# speedup vs baseline: 1.9258x; 1.9258x over previous
import functools

import jax
import jax.numpy as jnp
from jax import lax
from jax.experimental import pallas as pl
from jax.experimental.pallas import tpu as pltpu
from jax.experimental.pallas import tpu_sc as plsc

_NC = 2
_NS = 16
_NW = _NC * _NS
_GROUP = 128
_DC = 32


def _proj_body(feat_ref, wn_ref, ws_ref, b_ref, hext_ref, self_ref):
    f = feat_ref[...]
    dn = (((1,), (1,)), ((), ()))
    h = lax.dot_general(f, wn_ref[...], dn, preferred_element_type=jnp.float32)
    d = h.shape[1]
    hext_ref[:, :d] = h.astype(jnp.bfloat16)
    hext_ref[:, d:] = jnp.ones((f.shape[0], _DC), jnp.bfloat16)
    s = lax.dot_general(f, ws_ref[...], dn, preferred_element_type=jnp.float32)
    self_ref[...] = s + b_ref[...]


def _combine_body(n, d, parts_ref, self_ref, out_ref):
    p0 = parts_ref[0]
    summed = p0[:n, :d].astype(jnp.float32)
    cnt = p0[:n, d:d + 1].astype(jnp.float32)
    out_ref[...] = self_ref[...] + summed / jnp.maximum(cnt, 1.0)


def _sc_agg_body(n_acc, de, base, nextra, hext_hbm, edges_hbm,
                 out_hbm, ebuf, rows, zbuf, acc, ei0, ei1, ei2, ei3, ro0, ro1):
    sid = lax.axis_index("s")
    rows_per_tile = n_acc // _NS
    row0 = sid * rows_per_tile
    ei = (ei0, ei1, ei2, ei3)
    ro = (ro0, ro1)
    gstart = sid * base + 4 * jnp.minimum(sid, nextra)
    groups = base + jnp.where(sid < nextra, 4, 0)

    for q in range(4):
        pltpu.async_copy(edges_hbm.at[gstart + q], ebuf.at[q], ei[q])

    zrows = zbuf.shape[0]

    @pl.loop(0, zrows)
    def _(i):
        @pl.loop(0, de, step=32)
        def _(j):
            zbuf[i, pl.ds(j, 32)] = jnp.zeros((32,), jnp.bfloat16)

    @pl.loop(0, rows_per_tile, step=zrows)
    def _(r):
        pltpu.sync_copy(zbuf, acc.at[pl.ds(row0 + r, zrows)])

    plsc.subcore_barrier()

    for p in range(2):
        pltpu.make_async_copy(edges_hbm.at[gstart + p], ebuf.at[p],
                              ei[p]).wait()
        pltpu.async_copy(hext_hbm.at[ebuf.at[p, 0]], rows.at[p], ro[p])

    @pl.loop(0, groups, step=4)
    def _(g):
        for k in range(4):
            gi = g + k
            p, q = k % 2, k
            pltpu.make_async_copy(hext_hbm.at[ebuf.at[q, 0]], rows.at[p],
                                  ro[p]).wait()
            pltpu.sync_copy(rows.at[p], acc.at[ebuf.at[q, 1]], add=True)

            @pl.when(gi + 2 < groups)
            def _():
                q2 = (q + 2) % 4
                pltpu.make_async_copy(edges_hbm.at[gstart + gi + 2],
                                      ebuf.at[q2], ei[q2]).wait()
                pltpu.async_copy(hext_hbm.at[ebuf.at[q2, 0]], rows.at[p],
                                 ro[p])

            @pl.when(gi + 4 < groups)
            def _():
                pltpu.async_copy(edges_hbm.at[gstart + gi + 4],
                                 ebuf.at[q], ei[q])

    plsc.subcore_barrier()
    pltpu.sync_copy(acc.at[pl.ds(row0, rows_per_tile)],
                    out_hbm.at[0, pl.ds(row0, rows_per_tile)])


def kernel(feat, edge_index, W_neigh, W_self, b_self):
    n, d_in = feat.shape
    d_out = W_neigh.shape[0]
    e = edge_index.shape[1]
    de = d_out + _DC

    total_groups = -(-e // (_GROUP * 4)) * 4
    e_pad = total_groups * _GROUP
    base = (total_groups // _NS) // 4 * 4
    nextra = (total_groups - _NS * base) // 4
    n_acc = -(-(n + 1) // (_NS * 8)) * (_NS * 8)

    src = edge_index[0]
    dst = edge_index[1]
    pad = e_pad - e
    if pad:
        dum = n + jnp.arange(pad, dtype=jnp.int32) % (n_acc - n)
        src = jnp.concatenate([src, jnp.zeros((pad,), jnp.int32)])
        dst = jnp.concatenate([dst, dum])
    edges = jnp.stack([src.reshape(total_groups, _GROUP),
                       dst.reshape(total_groups, _GROUP)], axis=1)

    h_ext, rst_self = pl.pallas_call(
        _proj_body,
        out_shape=[
            jax.ShapeDtypeStruct((n, de), jnp.bfloat16),
            jax.ShapeDtypeStruct((n, d_out), jnp.float32),
        ],
    )(feat, W_neigh, W_self, b_self.reshape(1, d_out))

    sc_agg = pl.kernel(
        functools.partial(_sc_agg_body, n_acc, de, base, nextra),
        out_type=jax.ShapeDtypeStruct((1, n_acc, de), jnp.bfloat16),
        mesh=plsc.VectorSubcoreMesh(core_axis_name="c", subcore_axis_name="s",
                                    num_cores=1),
        compiler_params=pltpu.CompilerParams(use_tc_tiling_on_sc=False),
        scratch_types=[
            pltpu.VMEM((4, 2, _GROUP), jnp.int32),
            pltpu.VMEM((2, _GROUP, de), jnp.bfloat16),
            pltpu.VMEM((n_acc // _NS // 4, de), jnp.bfloat16),
            pltpu.VMEM_SHARED((n_acc, de), jnp.bfloat16),
        ] + [pltpu.SemaphoreType.DMA] * 6,
    )
    partials = sc_agg(h_ext, edges)

    rst = pl.pallas_call(
        functools.partial(_combine_body, n, d_out),
        out_shape=jax.ShapeDtypeStruct((n, d_out), jnp.float32),
    )(partials, rst_self)
    return rst
